# baseline (speedup 1.0000x reference)
import functools

import jax
import jax.numpy as jnp
from jax import lax
from jax.experimental import pallas as pl
from jax.experimental.pallas import tpu as pltpu
from jax.experimental.pallas import tpu_sc as plsc

B_, LQ, DIM = 16, 32, 128
ND, LD = 4096, 32
NT = ND * LD
NQ = ND // 4
QW = 4 * LD
R = B_ * LQ
TOPP, TOPK = 32, 10
TILE_T = 4096
TILE_D = TILE_T // LD
TILE_Q = TILE_D // 4
NTILES = NT // TILE_T
NEG = float("-inf")
BIGI = 2**31 - 1


def _a_body(q_ref, v_ref, sim_ref, dmax_ref, darg_ref, qmax_ref, qarg_ref):
    i = pl.program_id(0)
    q = q_ref[...]
    v = v_ref[...]
    dims = (((1,), (1,)), ((), ()))
    s = lax.dot_general(q, v, dims, preferred_element_type=jnp.float32)
    for g in range(TILE_Q):
        sim_ref[:, 0, g * 8:(g + 1) * 8, :] = (
            s[:, g * QW:(g + 1) * QW].reshape(R // 8, 8, QW))
    st = lax.dot_general(v, q, dims, preferred_element_type=jnp.float32)
    s3 = st.reshape(TILE_D, LD, R)
    dm = s3.max(axis=1)
    dmax_ref[...] = dm
    li = lax.broadcasted_iota(jnp.int32, (TILE_D, LD, R), 1)
    lmin = jnp.where(s3 == dm[:, None, :], li, BIGI).min(axis=1)
    dl = lax.broadcasted_iota(jnp.int32, (TILE_D, R), 0)
    da = i * TILE_T + dl * LD + lmin
    darg_ref[...] = da
    dm4 = dm.reshape(TILE_Q, 4, R)
    da4 = da.reshape(TILE_Q, 4, R)
    qm = dm4.max(axis=1)
    qmax_ref[...] = qm
    qarg_ref[...] = jnp.where(dm4 == qm[:, None, :], da4, BIGI).min(axis=1)


_stage_a = pl.pallas_call(
    _a_body,
    grid=(NTILES,),
    in_specs=[
        pl.BlockSpec((R, DIM), lambda i: (0, 0)),
        pl.BlockSpec((TILE_T, DIM), lambda i: (i, 0)),
    ],
    out_specs=[
        pl.BlockSpec((R // 8, 1, 8 * TILE_Q, QW), lambda i: (0, i, 0, 0)),
        pl.BlockSpec((TILE_D, R), lambda i: (i, 0)),
        pl.BlockSpec((TILE_D, R), lambda i: (i, 0)),
        pl.BlockSpec((TILE_Q, R), lambda i: (i, 0)),
        pl.BlockSpec((TILE_Q, R), lambda i: (i, 0)),
    ],
    out_shape=[
        jax.ShapeDtypeStruct((R // 8, NTILES, 8 * TILE_Q, QW), jnp.float32),
        jax.ShapeDtypeStruct((ND, R), jnp.float32),
        jax.ShapeDtypeStruct((ND, R), jnp.int32),
        jax.ShapeDtypeStruct((NQ, R), jnp.float32),
        jax.ShapeDtypeStruct((NQ, R), jnp.int32),
    ],
    compiler_params=pltpu.CompilerParams(
        dimension_semantics=("arbitrary",),
    ),
)


RB = 256
NRB = R // RB


def _b_body(qmax_ref, qarg_ref, ids_ref, vals_scr):
    vals_scr[...] = qmax_ref[...]
    args = qarg_ref[...]
    qio = lax.broadcasted_iota(jnp.int32, (RB, NQ), 1)
    jio = lax.broadcasted_iota(jnp.int32, (RB, TOPP), 1)

    def step(j, acc):
        vals = vals_scr[...]
        m = vals.max(axis=1, keepdims=True)
        a = jnp.where(vals == m, args, BIGI).min(axis=1, keepdims=True)
        qj = lax.shift_right_logical(a, 7)
        vals_scr[...] = jnp.where(qio == qj, NEG, vals)
        return jnp.where(jio == j, qj, acc)

    ids_ref[...] = lax.fori_loop(0, TOPP, step,
                                 jnp.zeros((RB, TOPP), jnp.int32))


_stage_b = pl.pallas_call(
    _b_body,
    grid=(NRB,),
    in_specs=[
        pl.BlockSpec((RB, NQ), lambda i: (i, 0)),
        pl.BlockSpec((RB, NQ), lambda i: (i, 0)),
    ],
    out_specs=pl.BlockSpec((RB, TOPP), lambda i: (i, 0)),
    out_shape=jax.ShapeDtypeStruct((R, TOPP), jnp.int32),
    scratch_shapes=[pltpu.VMEM((RB, NQ), jnp.float32)],
    compiler_params=pltpu.CompilerParams(
        dimension_semantics=("arbitrary",),
    ),
)


_NW = 32
_PER_W = (R * TOPP) // _NW
_CHUNK = 128
_NCH = _PER_W // _CHUNK


def _c_body(table_hbm, idx_hbm, out_hbm, idx_v, rows_v, sem):
    wid = lax.axis_index("s") * 2 + lax.axis_index("c")
    base = wid * _PER_W
    for c in range(_NCH):
        pltpu.sync_copy(idx_hbm.at[pl.ds(base + c * _CHUNK, _CHUNK)],
                        idx_v.at[c])
    copies = [pltpu.async_copy(table_hbm.at[idx_v.at[c]], rows_v.at[c], sem)
              for c in range(_NCH)]
    for c in range(_NCH):
        copies[c].wait()
        pltpu.sync_copy(rows_v.at[c],
                        out_hbm.at[pl.ds(base + c * _CHUNK, _CHUNK)])


@functools.cache
def _gather_sc_built():
    return functools.partial(
        pl.kernel,
        mesh=plsc.VectorSubcoreMesh(core_axis_name="c", subcore_axis_name="s"),
        out_type=jax.ShapeDtypeStruct((R * TOPP, QW), jnp.float32),
        scratch_types=[
            pltpu.VMEM((_NCH, _CHUNK), jnp.int32),
            pltpu.VMEM((_NCH, _CHUNK, QW), jnp.float32),
            pltpu.SemaphoreType.DMA,
        ],
    )(_c_body)


def _gather_sc(table, idx):
    return _gather_sc_built()(table, idx)


KPAD = 128
GW = TOPP * QW


RD = 256


def _d_body(gat_ref, qid_ref, dmax_ref, darg_ref, sc_ref, pid_ref, gv_scr):
    gv_scr[...] = gat_ref[...]
    qd = qid_ref[...]
    lio = lax.broadcasted_iota(jnp.int32, (RD, TOPP, QW), 2)
    gi = (qd[:, :, None] * QW + lio).reshape(RD, GW)

    def tstep(j, carry):
        gv = gv_scr[...]
        m = gv.max(axis=1, keepdims=True)
        a = jnp.where(gv == m, gi, BIGI).min(axis=1, keepdims=True)
        gv_scr[...] = jnp.where(gi == a, NEG, gv)
        return (m, a)

    tv, ti = lax.fori_loop(
        0, TOPP, tstep,
        (jnp.zeros((RD, 1), jnp.float32), jnp.zeros((RD, 1), jnp.int32)))
    dmax = dmax_ref[...]
    darg = darg_ref[...]
    ge = ((dmax > tv) | ((dmax == tv) & (darg <= ti))).astype(jnp.int32)
    nb = RD // LQ
    mask = ge.reshape(nb, LQ, ND).max(axis=1)
    s_all = dmax.reshape(nb, LQ, ND).sum(axis=1)
    s0 = jnp.where(mask > 0, s_all, NEG)
    pio = lax.broadcasted_iota(jnp.int32, (nb, ND), 1)
    kio = lax.broadcasted_iota(jnp.int32, (nb, KPAD), 1)

    def kstep(j, carry):
        s, accs, accp = carry
        m = s.max(axis=1, keepdims=True)
        pd = jnp.where(s == m, pio, BIGI).min(axis=1, keepdims=True)
        accs = jnp.where(kio == j, m, accs)
        accp = jnp.where(kio == j, jnp.where(m == NEG, -1, pd), accp)
        s = jnp.where(pio == pd, NEG, s)
        return (s, accs, accp)

    _, accs, accp = lax.fori_loop(
        0, TOPK, kstep,
        (s0, jnp.zeros((nb, KPAD), jnp.float32), jnp.zeros((nb, KPAD), jnp.int32)))
    sc_ref[...] = accs.reshape(RD // LQ, 1, KPAD)
    pid_ref[...] = accp.reshape(RD // LQ, 1, KPAD)


_stage_d = pl.pallas_call(
    _d_body,
    grid=(R // RD,),
    in_specs=[
        pl.BlockSpec((RD, GW), lambda i: (i, 0)),
        pl.BlockSpec((RD, TOPP), lambda i: (i, 0)),
        pl.BlockSpec((RD, ND), lambda i: (i, 0)),
        pl.BlockSpec((RD, ND), lambda i: (i, 0)),
    ],
    out_specs=[
        pl.BlockSpec((RD // LQ, 1, KPAD), lambda i: (i, 0, 0)),
        pl.BlockSpec((RD // LQ, 1, KPAD), lambda i: (i, 0, 0)),
    ],
    out_shape=[
        jax.ShapeDtypeStruct((B_, 1, KPAD), jnp.float32),
        jax.ShapeDtypeStruct((B_, 1, KPAD), jnp.int32),
    ],
    scratch_shapes=[pltpu.VMEM((RD, GW), jnp.float32)],
    compiler_params=pltpu.CompilerParams(
        dimension_semantics=("arbitrary",),
    ),
)


def kernel(q_vectors, vectors, emb2pid, p, k):
    q = jnp.where(jnp.isnan(q_vectors), 0.0, q_vectors)
    qs = q.reshape(R, DIM)
    sim, dmax_t, darg_t, qmax_t, qarg_t = _stage_a(qs, vectors)
    quad_ids = _stage_b(qmax_t.T, qarg_t.T)
    r_col = jnp.arange(R, dtype=jnp.int32)[:, None]
    c_idx = ((r_col >> 3) * (NQ * 8) + (quad_ids >> 4) * QW
             + (quad_ids & 15) * 8 + (r_col & 7)).reshape(-1)
    gathered = _gather_sc(sim.reshape(R * NQ, QW), c_idx)
    gat = gathered.reshape(R, GW)
    scores, pids = _stage_d(gat, quad_ids, dmax_t.T, darg_t.T)
    return scores[:, 0, :TOPK], pids[:, 0, :TOPK]


# speedup vs baseline: 1.0049x; 1.0049x over previous
import functools

import jax
import jax.numpy as jnp
from jax import lax
from jax.experimental import pallas as pl
from jax.experimental.pallas import tpu as pltpu
from jax.experimental.pallas import tpu_sc as plsc

B_, LQ, DIM = 16, 32, 128
ND, LD = 4096, 32
NT = ND * LD
NQ = ND // 4
QW = 4 * LD
R = B_ * LQ
TOPP, TOPK = 32, 10
TILE_T = 4096
TILE_D = TILE_T // LD
TILE_Q = TILE_D // 4
NTILES = NT // TILE_T
NEG = float("-inf")
BIGI = 2**31 - 1


def _a_body(q_ref, v_ref, sim_ref, dmax_ref, darg_ref, qmax_ref, qarg_ref):
    i = pl.program_id(0)
    q = q_ref[...]
    v = v_ref[...]
    dims = (((1,), (1,)), ((), ()))
    s = lax.dot_general(q, v, dims, preferred_element_type=jnp.float32)
    for g in range(TILE_Q):
        sim_ref[:, 0, g * 8:(g + 1) * 8, :] = (
            s[:, g * QW:(g + 1) * QW].reshape(R // 8, 8, QW))
    st = lax.dot_general(v, q, dims, preferred_element_type=jnp.float32)
    s3 = st.reshape(TILE_D, LD, R)
    dm = s3.max(axis=1)
    dmax_ref[...] = dm
    li = lax.broadcasted_iota(jnp.int32, (TILE_D, LD, R), 1)
    lmin = jnp.where(s3 == dm[:, None, :], li, BIGI).min(axis=1)
    dl = lax.broadcasted_iota(jnp.int32, (TILE_D, R), 0)
    da = i * TILE_T + dl * LD + lmin
    darg_ref[...] = da
    dm4 = dm.reshape(TILE_Q, 4, R)
    da4 = da.reshape(TILE_Q, 4, R)
    qm = dm4.max(axis=1)
    qmax_ref[...] = qm
    qarg_ref[...] = jnp.where(dm4 == qm[:, None, :], da4, BIGI).min(axis=1)


_stage_a = pl.pallas_call(
    _a_body,
    grid=(NTILES,),
    in_specs=[
        pl.BlockSpec((R, DIM), lambda i: (0, 0)),
        pl.BlockSpec((TILE_T, DIM), lambda i: (i, 0)),
    ],
    out_specs=[
        pl.BlockSpec((R // 8, 1, 8 * TILE_Q, QW), lambda i: (0, i, 0, 0)),
        pl.BlockSpec((TILE_D, R), lambda i: (i, 0)),
        pl.BlockSpec((TILE_D, R), lambda i: (i, 0)),
        pl.BlockSpec((TILE_Q, R), lambda i: (i, 0)),
        pl.BlockSpec((TILE_Q, R), lambda i: (i, 0)),
    ],
    out_shape=[
        jax.ShapeDtypeStruct((R // 8, NTILES, 8 * TILE_Q, QW), jnp.float32),
        jax.ShapeDtypeStruct((ND, R), jnp.float32),
        jax.ShapeDtypeStruct((ND, R), jnp.int32),
        jax.ShapeDtypeStruct((NQ, R), jnp.float32),
        jax.ShapeDtypeStruct((NQ, R), jnp.int32),
    ],
    compiler_params=pltpu.CompilerParams(
        dimension_semantics=("arbitrary",),
    ),
)


RB = 256
NRB = R // RB


def _b_body(qmax_ref, qarg_ref, ids_ref, vals_scr):
    vals_scr[...] = qmax_ref[...]
    args = qarg_ref[...]
    qio = lax.broadcasted_iota(jnp.int32, (RB, NQ), 1)
    jio = lax.broadcasted_iota(jnp.int32, (RB, TOPP), 1)

    def step(j, acc):
        vals = vals_scr[...]
        m = vals.max(axis=1, keepdims=True)
        a = jnp.where(vals == m, args, BIGI).min(axis=1, keepdims=True)
        qj = lax.shift_right_logical(a, 7)
        vals_scr[...] = jnp.where(qio == qj, NEG, vals)
        return jnp.where(jio == j, qj, acc)

    ids_ref[...] = lax.fori_loop(0, TOPP, step,
                                 jnp.zeros((RB, TOPP), jnp.int32))


_stage_b = pl.pallas_call(
    _b_body,
    grid=(NRB,),
    in_specs=[
        pl.BlockSpec((RB, NQ), lambda i: (i, 0)),
        pl.BlockSpec((RB, NQ), lambda i: (i, 0)),
    ],
    out_specs=pl.BlockSpec((RB, TOPP), lambda i: (i, 0)),
    out_shape=jax.ShapeDtypeStruct((R, TOPP), jnp.int32),
    scratch_shapes=[pltpu.VMEM((RB, NQ), jnp.float32)],
    compiler_params=pltpu.CompilerParams(
        dimension_semantics=("arbitrary",),
    ),
)


_NW = 32
_PER_W = (R * TOPP) // _NW
_CHUNK = 128
_NCH = _PER_W // _CHUNK


def _c_body(table_hbm, idx_hbm, out_hbm, idx_v, rows_v, sem):
    wid = lax.axis_index("s") * 2 + lax.axis_index("c")
    base = wid * _PER_W
    for c in range(_NCH):
        pltpu.sync_copy(idx_hbm.at[pl.ds(base + c * _CHUNK, _CHUNK)],
                        idx_v.at[c])
    copies = [pltpu.async_copy(table_hbm.at[idx_v.at[c]], rows_v.at[c], sem)
              for c in range(_NCH)]
    for c in range(_NCH):
        copies[c].wait()
        pltpu.sync_copy(rows_v.at[c],
                        out_hbm.at[pl.ds(base + c * _CHUNK, _CHUNK)])


@functools.cache
def _gather_sc_built():
    return functools.partial(
        pl.kernel,
        mesh=plsc.VectorSubcoreMesh(core_axis_name="c", subcore_axis_name="s"),
        out_type=jax.ShapeDtypeStruct((R * TOPP, QW), jnp.float32),
        scratch_types=[
            pltpu.VMEM((_NCH, _CHUNK), jnp.int32),
            pltpu.VMEM((_NCH, _CHUNK, QW), jnp.float32),
            pltpu.SemaphoreType.DMA,
        ],
    )(_c_body)


def _gather_sc(table, idx):
    return _gather_sc_built()(table, idx)


KPAD = 128
GW = TOPP * QW


RD = 512


def _d_body(gat_ref, qid_ref, dmax_ref, darg_ref, sc_ref, pid_ref, gv_scr):
    gv_scr[...] = gat_ref[...]
    qd = qid_ref[...]
    lio = lax.broadcasted_iota(jnp.int32, (RD, TOPP, QW), 2)
    gi = (qd[:, :, None] * QW + lio).reshape(RD, GW)

    def tstep(j, carry):
        gv = gv_scr[...]
        m = gv.max(axis=1, keepdims=True)
        a = jnp.where(gv == m, gi, BIGI).min(axis=1, keepdims=True)
        gv_scr[...] = jnp.where(gi == a, NEG, gv)
        return (m, a)

    tv, ti = lax.fori_loop(
        0, TOPP, tstep,
        (jnp.zeros((RD, 1), jnp.float32), jnp.zeros((RD, 1), jnp.int32)))
    dmax = dmax_ref[...]
    darg = darg_ref[...]
    ge = ((dmax > tv) | ((dmax == tv) & (darg <= ti))).astype(jnp.int32)
    nb = RD // LQ
    mask = ge.reshape(nb, LQ, ND).max(axis=1)
    s_all = dmax.reshape(nb, LQ, ND).sum(axis=1)
    s0 = jnp.where(mask > 0, s_all, NEG)
    pio = lax.broadcasted_iota(jnp.int32, (nb, ND), 1)
    kio = lax.broadcasted_iota(jnp.int32, (nb, KPAD), 1)

    def kstep(j, carry):
        s, accs, accp = carry
        m = s.max(axis=1, keepdims=True)
        pd = jnp.where(s == m, pio, BIGI).min(axis=1, keepdims=True)
        accs = jnp.where(kio == j, m, accs)
        accp = jnp.where(kio == j, jnp.where(m == NEG, -1, pd), accp)
        s = jnp.where(pio == pd, NEG, s)
        return (s, accs, accp)

    _, accs, accp = lax.fori_loop(
        0, TOPK, kstep,
        (s0, jnp.zeros((nb, KPAD), jnp.float32), jnp.zeros((nb, KPAD), jnp.int32)))
    sc_ref[...] = accs.reshape(RD // LQ, 1, KPAD)
    pid_ref[...] = accp.reshape(RD // LQ, 1, KPAD)


_stage_d = pl.pallas_call(
    _d_body,
    grid=(R // RD,),
    in_specs=[
        pl.BlockSpec((RD, GW), lambda i: (i, 0)),
        pl.BlockSpec((RD, TOPP), lambda i: (i, 0)),
        pl.BlockSpec((RD, ND), lambda i: (i, 0)),
        pl.BlockSpec((RD, ND), lambda i: (i, 0)),
    ],
    out_specs=[
        pl.BlockSpec((RD // LQ, 1, KPAD), lambda i: (i, 0, 0)),
        pl.BlockSpec((RD // LQ, 1, KPAD), lambda i: (i, 0, 0)),
    ],
    out_shape=[
        jax.ShapeDtypeStruct((B_, 1, KPAD), jnp.float32),
        jax.ShapeDtypeStruct((B_, 1, KPAD), jnp.int32),
    ],
    scratch_shapes=[pltpu.VMEM((RD, GW), jnp.float32)],
    compiler_params=pltpu.CompilerParams(
        dimension_semantics=("arbitrary",),
    ),
)


def kernel(q_vectors, vectors, emb2pid, p, k):
    q = jnp.where(jnp.isnan(q_vectors), 0.0, q_vectors)
    qs = q.reshape(R, DIM)
    sim, dmax_t, darg_t, qmax_t, qarg_t = _stage_a(qs, vectors)
    quad_ids = _stage_b(qmax_t.T, qarg_t.T)
    r_col = jnp.arange(R, dtype=jnp.int32)[:, None]
    c_idx = ((r_col >> 3) * (NQ * 8) + (quad_ids >> 4) * QW
             + (quad_ids & 15) * 8 + (r_col & 7)).reshape(-1)
    gathered = _gather_sc(sim.reshape(R * NQ, QW), c_idx)
    gat = gathered.reshape(R, GW)
    scores, pids = _stage_d(gat, quad_ids, dmax_t.T, darg_t.T)
    return scores[:, 0, :TOPK], pids[:, 0, :TOPK]
